# baseline (speedup 1.0000x reference)
import functools

import jax
import jax.numpy as jnp
from jax.experimental import pallas as pl
from jax.experimental.pallas import tpu as pltpu

_K = 4
_TOP_M = 8
_TEMP = 0.07
_EPS = 1e-06
_NUM_CLASSES = 1000
_TB = 2048


def _stats_for_batch(b, u, beta_row, coeff_ref, c_ref, *, K, T, top_m):
    plane = coeff_ref[b]
    gram = jax.lax.dot_general(
        u, u, (((1,), (1,)), ((), ())),
        preferred_element_type=jnp.float32,
        precision=jax.lax.Precision.HIGHEST,
    )
    row_i = jax.lax.broadcasted_iota(jnp.int32, (K, K), 0)
    col_j = jax.lax.broadcasted_iota(jnp.int32, (K, K), 1)
    lower = (col_j < row_i).astype(jnp.float32)
    L = lower * beta_row * gram
    eye = (col_j == row_i).astype(jnp.float32)
    hp = jax.lax.Precision.HIGHEST
    L2 = jax.lax.dot_general(L, L, (((1,), (0,)), ((), ())),
                             preferred_element_type=jnp.float32, precision=hp)
    L3 = jax.lax.dot_general(L2, L, (((1,), (0,)), ((), ())),
                             preferred_element_type=jnp.float32, precision=hp)
    M = eye - L + L2 - L3

    coeff = jnp.zeros_like(plane)
    for j in range(K):
        coeff = coeff + M[:, j:j + 1] * plane[j:j + 1, :]

    m = jnp.max(coeff, axis=1, keepdims=True)
    e = jnp.exp((coeff - m) * (1.0 / _TEMP))
    z = jnp.sum(e, axis=1, keepdims=True)
    s_soft = jnp.sum(e * coeff, axis=1, keepdims=True) / z

    iota = jax.lax.broadcasted_iota(jnp.int32, (K, T), 1)
    cur = coeff
    s_top = jnp.zeros((K, 1), dtype=jnp.float32)
    for _ in range(top_m):
        mx = jnp.max(cur, axis=1, keepdims=True)
        s_top = s_top + mx
        hit = jnp.where(cur == mx, iota, T)
        first = jnp.min(hit, axis=1, keepdims=True)
        cur = jnp.where(iota == first, jnp.float32(-jnp.inf), cur)

    c_ref[:, b:b + 1] = 0.5 * (s_top + s_soft)


def _fused_kernel(e_ref, u_ref, clsw_ref, clsb_ref, beta_ref, alpha_ref,
                  out_ref, coeff_ref, wu_ref, c_ref, *, B, T, K, top_m):
    i = pl.program_id(0)
    tblks = T // _TB
    a_steps = B * tblks
    u = u_ref[...]

    @pl.when(i < a_steps)
    def _():
        res = jax.lax.dot_general(
            u, e_ref[0], (((1,), (1,)), ((), ())),
            preferred_element_type=jnp.float32,
        )
        for s in range(a_steps):
            @pl.when(i == s)
            def _():
                b, tb = divmod(s, tblks)
                coeff_ref[b, :, tb * _TB:(tb + 1) * _TB] = res

    for s in range(K):
        @pl.when(i == s)
        def _():
            wu_ref[s:s + 1, :] = jax.lax.dot_general(
                u[s:s + 1], clsw_ref[0], (((1,), (1,)), ((), ())),
                preferred_element_type=jnp.float32,
            )

    for b in range(B):
        @pl.when(i == (b + 1) * tblks)
        def _():
            _stats_for_batch(b, u, beta_ref[...], coeff_ref, c_ref,
                             K=K, T=T, top_m=top_m)

    @pl.when(i == a_steps)
    def _():
        hp = jax.lax.Precision.HIGHEST
        ac = alpha_ref[...].reshape(K, 1) * c_ref[...]
        logits = jax.lax.dot_general(
            ac, wu_ref[...], (((0,), (0,)), ((), ())),
            preferred_element_type=jnp.float32, precision=hp,
        )
        bias = jax.lax.dot_general(
            alpha_ref[...], clsb_ref[...], (((1,), (0,)), ((), ())),
            preferred_element_type=jnp.float32, precision=hp,
        )
        out_ref[...] = logits + bias


def kernel(E, v, m_logits, cls_W, cls_b, beta, alpha):
    B, T, D = E.shape
    K = v.shape[0]
    top_m = min(_TOP_M, T)
    tblks = T // _TB
    a_steps = B * tblks

    mk = jax.nn.sigmoid(m_logits)
    vk = v * mk
    U = vk / (jnp.linalg.norm(vk, axis=1, keepdims=True) + _EPS)

    fused = functools.partial(_fused_kernel, B=B, T=T, K=K, top_m=top_m)
    logits = pl.pallas_call(
        fused,
        grid=(a_steps + 1,),
        in_specs=[
            pl.BlockSpec(
                (1, _TB, D),
                lambda i: (jnp.minimum(i, a_steps - 1) // tblks,
                           jnp.minimum(i, a_steps - 1) % tblks, 0)),
            pl.BlockSpec((K, D), lambda i: (0, 0)),
            pl.BlockSpec((1, _NUM_CLASSES, D),
                         lambda i: (jnp.minimum(i, K - 1), 0, 0)),
            pl.BlockSpec((K, _NUM_CLASSES), lambda i: (0, 0)),
            pl.BlockSpec((1, K), lambda i: (0, 0)),
            pl.BlockSpec((1, K), lambda i: (0, 0)),
        ],
        out_specs=pl.BlockSpec((B, _NUM_CLASSES), lambda i: (0, 0)),
        out_shape=jax.ShapeDtypeStruct((B, _NUM_CLASSES), jnp.float32),
        scratch_shapes=[
            pltpu.VMEM((B, K, T), jnp.float32),
            pltpu.VMEM((K, _NUM_CLASSES), jnp.float32),
            pltpu.VMEM((K, B), jnp.float32),
        ],
    )(E, U, cls_W, cls_b, beta.reshape(1, K), alpha.reshape(1, K))
    return logits


# speedup vs baseline: 1.0644x; 1.0644x over previous
import functools

import jax
import jax.numpy as jnp
from jax.experimental import pallas as pl
from jax.experimental.pallas import tpu as pltpu

_K = 4
_TOP_M = 8
_TEMP = 0.07
_EPS = 1e-06
_NUM_CLASSES = 1000
_TB = 4096


def _stats_for_batch(b, u, beta_row, coeff_ref, c_ref, *, K, T, top_m):
    plane = coeff_ref[b]
    gram = jax.lax.dot_general(
        u, u, (((1,), (1,)), ((), ())),
        preferred_element_type=jnp.float32,
        precision=jax.lax.Precision.HIGHEST,
    )
    row_i = jax.lax.broadcasted_iota(jnp.int32, (K, K), 0)
    col_j = jax.lax.broadcasted_iota(jnp.int32, (K, K), 1)
    lower = (col_j < row_i).astype(jnp.float32)
    L = lower * beta_row * gram
    eye = (col_j == row_i).astype(jnp.float32)
    hp = jax.lax.Precision.HIGHEST
    L2 = jax.lax.dot_general(L, L, (((1,), (0,)), ((), ())),
                             preferred_element_type=jnp.float32, precision=hp)
    L3 = jax.lax.dot_general(L2, L, (((1,), (0,)), ((), ())),
                             preferred_element_type=jnp.float32, precision=hp)
    M = eye - L + L2 - L3

    coeff = jnp.zeros_like(plane)
    for j in range(K):
        coeff = coeff + M[:, j:j + 1] * plane[j:j + 1, :]

    m = jnp.max(coeff, axis=1, keepdims=True)
    e = jnp.exp((coeff - m) * (1.0 / _TEMP))
    z = jnp.sum(e, axis=1, keepdims=True)
    s_soft = jnp.sum(e * coeff, axis=1, keepdims=True) / z

    iota = jax.lax.broadcasted_iota(jnp.int32, (K, T), 1)
    cur = coeff
    s_top = jnp.zeros((K, 1), dtype=jnp.float32)
    for _ in range(top_m):
        mx = jnp.max(cur, axis=1, keepdims=True)
        s_top = s_top + mx
        hit = jnp.where(cur == mx, iota, T)
        first = jnp.min(hit, axis=1, keepdims=True)
        cur = jnp.where(iota == first, jnp.float32(-jnp.inf), cur)

    c_ref[:, b:b + 1] = 0.5 * (s_top + s_soft)


def _fused_kernel(e_ref, u_ref, clsw_ref, clsb_ref, beta_ref, alpha_ref,
                  out_ref, coeff_ref, wu_ref, c_ref, *, B, T, K, top_m):
    i = pl.program_id(0)
    tblks = T // _TB
    a_steps = B * tblks
    u = u_ref[...]

    @pl.when(i < a_steps)
    def _():
        res = jax.lax.dot_general(
            u, e_ref[0], (((1,), (1,)), ((), ())),
            preferred_element_type=jnp.float32,
        )
        for s in range(a_steps):
            @pl.when(i == s)
            def _():
                b, tb = divmod(s, tblks)
                coeff_ref[b, :, tb * _TB:(tb + 1) * _TB] = res

    for s in range(K):
        @pl.when(i == s)
        def _():
            wu_ref[s:s + 1, :] = jax.lax.dot_general(
                u[s:s + 1], clsw_ref[0], (((1,), (1,)), ((), ())),
                preferred_element_type=jnp.float32,
            )

    for b in range(B):
        @pl.when(i == (b + 1) * tblks)
        def _():
            _stats_for_batch(b, u, beta_ref[...], coeff_ref, c_ref,
                             K=K, T=T, top_m=top_m)

    @pl.when(i == a_steps)
    def _():
        hp = jax.lax.Precision.HIGHEST
        ac = alpha_ref[...].reshape(K, 1) * c_ref[...]
        logits = jax.lax.dot_general(
            ac, wu_ref[...], (((0,), (0,)), ((), ())),
            preferred_element_type=jnp.float32, precision=hp,
        )
        bias = jax.lax.dot_general(
            alpha_ref[...], clsb_ref[...], (((1,), (0,)), ((), ())),
            preferred_element_type=jnp.float32, precision=hp,
        )
        out_ref[...] = logits + bias


def kernel(E, v, m_logits, cls_W, cls_b, beta, alpha):
    B, T, D = E.shape
    K = v.shape[0]
    top_m = min(_TOP_M, T)
    tblks = T // _TB
    a_steps = B * tblks

    mk = jax.nn.sigmoid(m_logits)
    vk = v * mk
    U = vk / (jnp.linalg.norm(vk, axis=1, keepdims=True) + _EPS)

    fused = functools.partial(_fused_kernel, B=B, T=T, K=K, top_m=top_m)
    logits = pl.pallas_call(
        fused,
        grid=(a_steps + 1,),
        in_specs=[
            pl.BlockSpec(
                (1, _TB, D),
                lambda i: (jnp.minimum(i, a_steps - 1) // tblks,
                           jnp.minimum(i, a_steps - 1) % tblks, 0)),
            pl.BlockSpec((K, D), lambda i: (0, 0)),
            pl.BlockSpec((1, _NUM_CLASSES, D),
                         lambda i: (jnp.minimum(i, K - 1), 0, 0)),
            pl.BlockSpec((K, _NUM_CLASSES), lambda i: (0, 0)),
            pl.BlockSpec((1, K), lambda i: (0, 0)),
            pl.BlockSpec((1, K), lambda i: (0, 0)),
        ],
        out_specs=pl.BlockSpec((B, _NUM_CLASSES), lambda i: (0, 0)),
        out_shape=jax.ShapeDtypeStruct((B, _NUM_CLASSES), jnp.float32),
        scratch_shapes=[
            pltpu.VMEM((B, K, T), jnp.float32),
            pltpu.VMEM((K, _NUM_CLASSES), jnp.float32),
            pltpu.VMEM((K, B), jnp.float32),
        ],
    )(E, U, cls_W, cls_b, beta.reshape(1, K), alpha.reshape(1, K))
    return logits
